# baseline (speedup 1.0000x reference)
import functools

import jax
import jax.numpy as jnp
from jax import lax
from jax.experimental import pallas as pl
from jax.experimental.pallas import tpu as pltpu
from jax.experimental.pallas import tpu_sc as plsc

N = 10000
E = 320000
DE = 16
DF = 128

NC, NS = 2, 16
NW = NC * NS
TCOLS = E // 128
CH_C = 10
CH_E = CH_C * 128
NCHUNK = TCOLS // CH_C
MAX_R = -(-NCHUNK // NW)
N_STRIPE = 624
N_TAIL = N - N_STRIPE * NS

_mesh = plsc.VectorSubcoreMesh(core_axis_name="c", subcore_axis_name="s")


@functools.partial(
    pl.kernel,
    out_type=jax.ShapeDtypeStruct((NC, N, DE), jnp.float32),
    mesh=_mesh,
    scratch_types=[
        pltpu.VMEM((2, CH_E), jnp.int32),
        pltpu.VMEM((2, 2, CH_C, 8, 128), jnp.float32),
        pltpu.VMEM((2, CH_E, DE), jnp.float32),
        pltpu.VMEM_SHARED((N, DE), jnp.float32),
        pltpu.SemaphoreType.DMA,
        pltpu.SemaphoreType.DMA,
        pltpu.SemaphoreType.DMA,
    ],
    compiler_params=pltpu.CompilerParams(use_tc_tiling_on_sc=False,
                                         needs_layout_passes=False),
)
def _sc_agg(ei_hbm, tiles_hbm, zeros_hbm, out_hbm, idx_v, colst_v, rows_v,
            shared, sem_l0, sem_l1, sem_s):
    cid = lax.axis_index("c")
    sid = lax.axis_index("s")
    wid = cid * NS + sid

    pltpu.sync_copy(zeros_hbm.at[pl.ds(sid * N_STRIPE, N_STRIPE)],
                    shared.at[pl.ds(sid * N_STRIPE, N_STRIPE)])

    @pl.when(sid == NS - 1)
    def _zero_tail():
        pltpu.sync_copy(zeros_hbm.at[pl.ds(N_STRIPE * NS, N_TAIL)],
                        shared.at[pl.ds(N_STRIPE * NS, N_TAIL)])

    plsc.subcore_barrier()

    recv = ei_hbm.at[1]
    sems = (sem_l0, sem_l1)
    iota16 = lax.iota(jnp.int32, 16)
    fhi = iota16 // 8
    flo = iota16 % 8

    def _start_load(ch, buf):
        pltpu.async_copy(recv.at[pl.ds(ch * CH_E, CH_E)],
                         idx_v.at[buf], sems[buf])
        pltpu.async_copy(tiles_hbm.at[:, pl.ds(ch * CH_C, CH_C)],
                         colst_v.at[buf], sems[buf])

    def _wait_load(buf):
        pltpu.make_async_copy(recv.at[pl.ds(0, CH_E)],
                              idx_v.at[buf], sems[buf]).wait()
        pltpu.make_async_copy(tiles_hbm.at[:, pl.ds(0, CH_C)],
                              colst_v.at[buf], sems[buf]).wait()

    def _transpose(buf):
        colst_b = colst_v.at[buf]
        rows_b = rows_v.at[buf]
        for tc in range(CH_C):
            tc_vec = jnp.full((16,), tc, jnp.int32)

            @plsc.parallel_loop(0, 128, unroll=8)
            def _t(d):
                vec = plsc.load_gather(
                    colst_b, [fhi, tc_vec, flo, jnp.full((16,), d, jnp.int32)])
                rows_b[tc * 128 + d] = vec

    def _fire_scatters(buf):
        descs = []
        for j in range(CH_C):
            descs.append(pltpu.async_copy(
                rows_v.at[buf].at[pl.ds(j * 128, 128)],
                shared.at[idx_v.at[buf].at[pl.ds(j * 128, 128)]],
                sem_s, add=True))
        return descs

    def _drain(descs):
        for d in descs:
            d.wait()

    def _valid(r):
        return wid + NW * r < NCHUNK

    _start_load(wid, 0)
    _start_load(wid + NW, 1)

    @pl.loop(0, MAX_R, step=2)
    def _pair(k):
        _wait_load(0)
        _transpose(0)
        d0 = _fire_scatters(0)

        @pl.when(_valid(k + 1))
        def _odd():
            _wait_load(1)
            _transpose(1)

        _drain(d0)

        @pl.when(_valid(k + 2))
        def _prefetch0():
            _start_load(wid + NW * (k + 2), 0)

        @pl.when(_valid(k + 1))
        def _odd_scatter():
            d1 = _fire_scatters(1)
            _drain(d1)

            @pl.when(_valid(k + 3))
            def _prefetch1():
                _start_load(wid + NW * (k + 3), 1)

    plsc.subcore_barrier()
    pltpu.sync_copy(shared.at[pl.ds(sid * N_STRIPE, N_STRIPE)],
                    out_hbm.at[cid].at[pl.ds(sid * N_STRIPE, N_STRIPE)])

    @pl.when(sid == NS - 1)
    def _out_tail():
        pltpu.sync_copy(shared.at[pl.ds(N_STRIPE * NS, N_TAIL)],
                        out_hbm.at[cid].at[pl.ds(N_STRIPE * NS, N_TAIL)])


_RB = 2000


def _mlp_body(x_ref, p_ref, wx_ref, we_ref, b_ref, o_ref):
    agg = p_ref[0] + p_ref[1]
    o_ref[...] = (
        jnp.dot(x_ref[...], wx_ref[...], preferred_element_type=jnp.float32)
        + jnp.dot(agg, we_ref[...], preferred_element_type=jnp.float32)
        + b_ref[...]
    )


def _mlp(x, parts, wx, we, b2):
    return pl.pallas_call(
        _mlp_body,
        grid=(N // _RB,),
        in_specs=[
            pl.BlockSpec((_RB, DF), lambda i: (i, 0)),
            pl.BlockSpec((NC, _RB, DE), lambda i: (0, i, 0)),
            pl.BlockSpec((DF, DF), lambda i: (0, 0)),
            pl.BlockSpec((DE, DF), lambda i: (0, 0)),
            pl.BlockSpec((1, DF), lambda i: (0, 0)),
        ],
        out_specs=pl.BlockSpec((_RB, DF), lambda i: (i, 0)),
        out_shape=jax.ShapeDtypeStruct((N, DF), jnp.float32),
    )(x, parts, wx, we, b2)


def kernel(x, edge_index, edge_attr, pos, W, b):
    zeros = jnp.zeros((N, DE), jnp.float32)
    tiles4 = edge_attr.T.reshape(NC, 8, TCOLS, 128).transpose(0, 2, 1, 3)
    parts = _sc_agg(edge_index, tiles4, zeros)
    x_ = _mlp(x, parts, W[:DF], W[DF:], b.reshape(1, DF))
    return (x_, edge_attr, edge_index, pos)


# speedup vs baseline: 1.0087x; 1.0087x over previous
import functools

import jax
import jax.numpy as jnp
from jax import lax
from jax.experimental import pallas as pl
from jax.experimental.pallas import tpu as pltpu
from jax.experimental.pallas import tpu_sc as plsc

N = 10000
E = 320000
DE = 16
DF = 128

NC, NS = 2, 16
NW = NC * NS
TCOLS = E // 128
CH_C = 10
CH_E = CH_C * 128
NCHUNK = TCOLS // CH_C
MAX_R = -(-NCHUNK // NW)
N_STRIPE = 624
N_TAIL = N - N_STRIPE * NS

_mesh = plsc.VectorSubcoreMesh(core_axis_name="c", subcore_axis_name="s")


@functools.partial(
    pl.kernel,
    out_type=jax.ShapeDtypeStruct((NC, N, DE), jnp.float32),
    mesh=_mesh,
    scratch_types=[
        pltpu.VMEM((2, CH_E), jnp.int32),
        pltpu.VMEM((2, 2 * CH_C * 1024), jnp.float32),
        pltpu.VMEM((2, CH_E, DE), jnp.float32),
        pltpu.VMEM_SHARED((N, DE), jnp.float32),
        pltpu.SemaphoreType.DMA,
        pltpu.SemaphoreType.DMA,
        pltpu.SemaphoreType.DMA,
    ],
    compiler_params=pltpu.CompilerParams(use_tc_tiling_on_sc=False,
                                         needs_layout_passes=False),
)
def _sc_agg(ei_hbm, tiles_hbm, zeros_hbm, out_hbm, idx_v, colst_v, rows_v,
            shared, sem_l0, sem_l1, sem_s):
    cid = lax.axis_index("c")
    sid = lax.axis_index("s")
    wid = cid * NS + sid

    pltpu.sync_copy(zeros_hbm.at[pl.ds(sid * N_STRIPE, N_STRIPE)],
                    shared.at[pl.ds(sid * N_STRIPE, N_STRIPE)])

    @pl.when(sid == NS - 1)
    def _zero_tail():
        pltpu.sync_copy(zeros_hbm.at[pl.ds(N_STRIPE * NS, N_TAIL)],
                        shared.at[pl.ds(N_STRIPE * NS, N_TAIL)])

    plsc.subcore_barrier()

    recv = ei_hbm.at[1]
    sems = (sem_l0, sem_l1)
    iota16 = lax.iota(jnp.int32, 16)
    fhi = iota16 // 8
    flo = iota16 % 8

    HALF = CH_C * 1024

    def _start_load(ch, buf):
        pltpu.async_copy(recv.at[pl.ds(ch * CH_E, CH_E)],
                         idx_v.at[buf], sems[buf])
        pltpu.async_copy(tiles_hbm.at[0].at[pl.ds(ch * HALF, HALF)],
                         colst_v.at[buf].at[pl.ds(0, HALF)], sems[buf])
        pltpu.async_copy(tiles_hbm.at[1].at[pl.ds(ch * HALF, HALF)],
                         colst_v.at[buf].at[pl.ds(HALF, HALF)], sems[buf])

    def _wait_load(buf):
        pltpu.make_async_copy(recv.at[pl.ds(0, CH_E)],
                              idx_v.at[buf], sems[buf]).wait()
        pltpu.make_async_copy(tiles_hbm.at[0].at[pl.ds(0, 2 * HALF)],
                              colst_v.at[buf], sems[buf]).wait()

    _cvec = [fhi * HALF + flo * 128 + tc * 1024 for tc in range(CH_C)]

    def _transpose(buf):
        colst_b = colst_v.at[buf]
        rows_b = rows_v.at[buf]
        for tc in range(CH_C):
            cvec_tc = _cvec[tc]

            @plsc.parallel_loop(0, 128, unroll=8)
            def _t(d):
                vec = plsc.load_gather(
                    colst_b, [cvec_tc + jnp.full((16,), d, jnp.int32)])
                rows_b[tc * 128 + d] = vec

    def _fire_scatters(buf):
        descs = []
        for j in range(CH_C):
            descs.append(pltpu.async_copy(
                rows_v.at[buf].at[pl.ds(j * 128, 128)],
                shared.at[idx_v.at[buf].at[pl.ds(j * 128, 128)]],
                sem_s, add=True))
        return descs

    def _drain(descs):
        for d in descs:
            d.wait()

    def _valid(r):
        return wid + NW * r < NCHUNK

    _start_load(wid, 0)
    _start_load(wid + NW, 1)

    @pl.loop(0, MAX_R, step=2)
    def _pair(k):
        _wait_load(0)
        _transpose(0)
        d0 = _fire_scatters(0)

        @pl.when(_valid(k + 1))
        def _odd():
            _wait_load(1)
            _transpose(1)

        _drain(d0)

        @pl.when(_valid(k + 2))
        def _prefetch0():
            _start_load(wid + NW * (k + 2), 0)

        @pl.when(_valid(k + 1))
        def _odd_scatter():
            d1 = _fire_scatters(1)
            _drain(d1)

            @pl.when(_valid(k + 3))
            def _prefetch1():
                _start_load(wid + NW * (k + 3), 1)

    plsc.subcore_barrier()
    pltpu.sync_copy(shared.at[pl.ds(sid * N_STRIPE, N_STRIPE)],
                    out_hbm.at[cid].at[pl.ds(sid * N_STRIPE, N_STRIPE)])

    @pl.when(sid == NS - 1)
    def _out_tail():
        pltpu.sync_copy(shared.at[pl.ds(N_STRIPE * NS, N_TAIL)],
                        out_hbm.at[cid].at[pl.ds(N_STRIPE * NS, N_TAIL)])


_RB = 2000


def _mlp_body(x_ref, p_ref, wx_ref, we_ref, b_ref, o_ref):
    agg = p_ref[0] + p_ref[1]
    o_ref[...] = (
        jnp.dot(x_ref[...], wx_ref[...], preferred_element_type=jnp.float32)
        + jnp.dot(agg, we_ref[...], preferred_element_type=jnp.float32)
        + b_ref[...]
    )


def _mlp(x, parts, wx, we, b2):
    return pl.pallas_call(
        _mlp_body,
        grid=(N // _RB,),
        in_specs=[
            pl.BlockSpec((_RB, DF), lambda i: (i, 0)),
            pl.BlockSpec((NC, _RB, DE), lambda i: (0, i, 0)),
            pl.BlockSpec((DF, DF), lambda i: (0, 0)),
            pl.BlockSpec((DE, DF), lambda i: (0, 0)),
            pl.BlockSpec((1, DF), lambda i: (0, 0)),
        ],
        out_specs=pl.BlockSpec((_RB, DF), lambda i: (i, 0)),
        out_shape=jax.ShapeDtypeStruct((N, DF), jnp.float32),
    )(x, parts, wx, we, b2)


def kernel(x, edge_index, edge_attr, pos, W, b):
    zeros = jnp.zeros((N, DE), jnp.float32)
    tiles4 = edge_attr.T.reshape(2, 8, TCOLS, 128).transpose(0, 2, 1, 3)
    tiles2 = tiles4.reshape(2, TCOLS * 1024)
    parts = _sc_agg(edge_index, tiles2, zeros)
    x_ = _mlp(x, parts, W[:DF], W[DF:], b.reshape(1, DF))
    return (x_, edge_attr, edge_index, pos)


# speedup vs baseline: 1.3724x; 1.3605x over previous
import functools

import jax
import jax.numpy as jnp
from jax import lax
from jax.experimental import pallas as pl
from jax.experimental.pallas import tpu as pltpu
from jax.experimental.pallas import tpu_sc as plsc

N = 10000
E = 320000
DE = 16
DF = 128

NC, NS = 2, 16
NW = NC * NS
PER_TILE = E // NW
CH_E = 1000
NCHUNK = PER_TILE // CH_E
N_STRIPE = 624
N_TAIL = N - N_STRIPE * NS

_BATCHES = [(j * 128, 128) for j in range(7)] + [(896, 104)]
assert sum(bi for _, bi in _BATCHES) == CH_E

_mesh = plsc.VectorSubcoreMesh(core_axis_name="c", subcore_axis_name="s")


@functools.partial(
    pl.kernel,
    out_type=jax.ShapeDtypeStruct((NC, N, DE), jnp.float32),
    mesh=_mesh,
    scratch_types=[
        pltpu.VMEM((2, CH_E), jnp.int32),
        pltpu.VMEM((2, DE, CH_E), jnp.float32),
        pltpu.VMEM((2, CH_E, DE), jnp.float32),
        pltpu.VMEM_SHARED((N, DE), jnp.float32),
        pltpu.SemaphoreType.DMA,
        pltpu.SemaphoreType.DMA,
        pltpu.SemaphoreType.DMA,
    ],
    compiler_params=pltpu.CompilerParams(use_tc_tiling_on_sc=False,
                                         needs_layout_passes=False),
)
def _sc_agg(ei_hbm, attrt_hbm, zeros_hbm, out_hbm, idx_v, colst_v, rows_v,
            shared, sem_l0, sem_l1, sem_s):
    cid = lax.axis_index("c")
    sid = lax.axis_index("s")
    wid = cid * NS + sid

    pltpu.sync_copy(zeros_hbm.at[pl.ds(sid * N_STRIPE, N_STRIPE)],
                    shared.at[pl.ds(sid * N_STRIPE, N_STRIPE)])

    @pl.when(sid == NS - 1)
    def _zero_tail():
        pltpu.sync_copy(zeros_hbm.at[pl.ds(N_STRIPE * NS, N_TAIL)],
                        shared.at[pl.ds(N_STRIPE * NS, N_TAIL)])

    plsc.subcore_barrier()

    edge_base = wid * PER_TILE
    recv = ei_hbm.at[1]
    sems = (sem_l0, sem_l1)
    iota16 = lax.iota(jnp.int32, 16)

    def _start_load(c, buf):
        pltpu.async_copy(recv.at[pl.ds(edge_base + c * CH_E, CH_E)],
                         idx_v.at[buf], sems[buf])
        pltpu.async_copy(attrt_hbm.at[:, pl.ds(edge_base + c * CH_E, CH_E)],
                         colst_v.at[buf], sems[buf])

    def _wait_load(buf):
        pltpu.make_async_copy(recv.at[pl.ds(0, CH_E)],
                              idx_v.at[buf], sems[buf]).wait()
        pltpu.make_async_copy(attrt_hbm.at[:, pl.ds(0, CH_E)],
                              colst_v.at[buf], sems[buf]).wait()

    def _transpose(buf):
        colst_b = colst_v.at[buf]
        rows_b = rows_v.at[buf]

        @plsc.parallel_loop(0, CH_E, unroll=8)
        def _t(e):
            vec = plsc.load_gather(colst_b,
                                   [iota16, jnp.full((16,), e, jnp.int32)])
            rows_b[e] = vec

    def _fire_scatters(buf):
        descs = []
        for off, bi in _BATCHES:
            descs.append(pltpu.async_copy(
                rows_v.at[buf].at[pl.ds(off, bi)],
                shared.at[idx_v.at[buf].at[pl.ds(off, bi)]],
                sem_s, add=True))
        return descs

    def _drain(descs):
        for d in descs:
            d.wait()

    _start_load(0, 0)
    _start_load(1, 1)

    @pl.loop(0, NCHUNK, step=2)
    def _pair(c):
        _wait_load(0)
        _transpose(0)
        d0 = _fire_scatters(0)
        _wait_load(1)
        _transpose(1)
        _drain(d0)

        @pl.when(c + 2 < NCHUNK)
        def _prefetch0():
            _start_load(c + 2, 0)

        d1 = _fire_scatters(1)
        _drain(d1)

        @pl.when(c + 3 < NCHUNK)
        def _prefetch1():
            _start_load(c + 3, 1)

    plsc.subcore_barrier()
    pltpu.sync_copy(shared.at[pl.ds(sid * N_STRIPE, N_STRIPE)],
                    out_hbm.at[cid].at[pl.ds(sid * N_STRIPE, N_STRIPE)])

    @pl.when(sid == NS - 1)
    def _out_tail():
        pltpu.sync_copy(shared.at[pl.ds(N_STRIPE * NS, N_TAIL)],
                        out_hbm.at[cid].at[pl.ds(N_STRIPE * NS, N_TAIL)])


_RB = 2000


def _mlp_body(x_ref, p_ref, wx_ref, we_ref, b_ref, o_ref):
    agg = p_ref[0] + p_ref[1]
    o_ref[...] = (
        jnp.dot(x_ref[...], wx_ref[...], preferred_element_type=jnp.float32)
        + jnp.dot(agg, we_ref[...], preferred_element_type=jnp.float32)
        + b_ref[...]
    )


def _mlp(x, parts, wx, we, b2):
    return pl.pallas_call(
        _mlp_body,
        grid=(N // _RB,),
        in_specs=[
            pl.BlockSpec((_RB, DF), lambda i: (i, 0)),
            pl.BlockSpec((NC, _RB, DE), lambda i: (0, i, 0)),
            pl.BlockSpec((DF, DF), lambda i: (0, 0)),
            pl.BlockSpec((DE, DF), lambda i: (0, 0)),
            pl.BlockSpec((1, DF), lambda i: (0, 0)),
        ],
        out_specs=pl.BlockSpec((_RB, DF), lambda i: (i, 0)),
        out_shape=jax.ShapeDtypeStruct((N, DF), jnp.float32),
    )(x, parts, wx, we, b2)


def kernel(x, edge_index, edge_attr, pos, W, b):
    zeros = jnp.zeros((N, DE), jnp.float32)
    parts = _sc_agg(edge_index, edge_attr.T, zeros)
    x_ = _mlp(x, parts, W[:DF], W[DF:], b.reshape(1, DF))
    return (x_, edge_attr, edge_index, pos)


# speedup vs baseline: 1.3962x; 1.0174x over previous
import functools

import jax
import jax.numpy as jnp
from jax import lax
from jax.experimental import pallas as pl
from jax.experimental.pallas import tpu as pltpu
from jax.experimental.pallas import tpu_sc as plsc

N = 10000
E = 320000
DE = 16
DF = 128

NC, NS = 2, 16
NW = NC * NS
PER_TILE = E // NW
CH_E = 1000
NCHUNK = PER_TILE // CH_E
N_STRIPE = 624
N_TAIL = N - N_STRIPE * NS

_BATCHES = [(j * 128, 128) for j in range(7)] + [(896, 104)]
assert sum(bi for _, bi in _BATCHES) == CH_E

_mesh = plsc.VectorSubcoreMesh(core_axis_name="c", subcore_axis_name="s")


@functools.partial(
    pl.kernel,
    out_type=jax.ShapeDtypeStruct((NC, N, DE), jnp.float32),
    mesh=_mesh,
    scratch_types=[
        pltpu.VMEM((2, CH_E), jnp.int32),
        pltpu.VMEM((2, DE, CH_E), jnp.float32),
        pltpu.VMEM((2, CH_E, DE), jnp.float32),
        pltpu.VMEM_SHARED((N, DE), jnp.float32),
        pltpu.SemaphoreType.DMA,
        pltpu.SemaphoreType.DMA,
        pltpu.SemaphoreType.DMA,
    ],
    compiler_params=pltpu.CompilerParams(use_tc_tiling_on_sc=False,
                                         needs_layout_passes=False),
)
def _sc_agg(ei_hbm, attrt_hbm, zeros_hbm, out_hbm, idx_v, colst_v, rows_v,
            shared, sem_l0, sem_l1, sem_s):
    cid = lax.axis_index("c")
    sid = lax.axis_index("s")
    wid = cid * NS + sid

    edge_base = wid * PER_TILE
    recv = ei_hbm.at[1]
    sems = (sem_l0, sem_l1)
    iota16 = lax.iota(jnp.int32, 16)

    def _start_load(c, buf):
        pltpu.async_copy(recv.at[pl.ds(edge_base + c * CH_E, CH_E)],
                         idx_v.at[buf], sems[buf])
        pltpu.async_copy(attrt_hbm.at[:, pl.ds(edge_base + c * CH_E, CH_E)],
                         colst_v.at[buf], sems[buf])

    _start_load(0, 0)
    _start_load(1, 1)

    pltpu.sync_copy(zeros_hbm.at[pl.ds(sid * N_STRIPE, N_STRIPE)],
                    shared.at[pl.ds(sid * N_STRIPE, N_STRIPE)])

    @pl.when(sid == NS - 1)
    def _zero_tail():
        pltpu.sync_copy(zeros_hbm.at[pl.ds(N_STRIPE * NS, N_TAIL)],
                        shared.at[pl.ds(N_STRIPE * NS, N_TAIL)])

    plsc.subcore_barrier()

    def _wait_load(buf):
        pltpu.make_async_copy(recv.at[pl.ds(0, CH_E)],
                              idx_v.at[buf], sems[buf]).wait()
        pltpu.make_async_copy(attrt_hbm.at[:, pl.ds(0, CH_E)],
                              colst_v.at[buf], sems[buf]).wait()

    def _transpose(buf):
        colst_b = colst_v.at[buf]
        rows_b = rows_v.at[buf]

        @plsc.parallel_loop(0, CH_E, unroll=8)
        def _t(e):
            vec = plsc.load_gather(colst_b,
                                   [iota16, jnp.full((16,), e, jnp.int32)])
            rows_b[e] = vec

    def _fire_scatters(buf):
        descs = []
        for off, bi in _BATCHES:
            descs.append(pltpu.async_copy(
                rows_v.at[buf].at[pl.ds(off, bi)],
                shared.at[idx_v.at[buf].at[pl.ds(off, bi)]],
                sem_s, add=True))
        return descs

    def _drain(descs):
        for d in descs:
            d.wait()

    @pl.loop(0, NCHUNK, step=2)
    def _pair(c):
        _wait_load(0)
        _transpose(0)
        d0 = _fire_scatters(0)
        _wait_load(1)
        _transpose(1)
        _drain(d0)

        @pl.when(c + 2 < NCHUNK)
        def _prefetch0():
            _start_load(c + 2, 0)

        d1 = _fire_scatters(1)
        _drain(d1)

        @pl.when(c + 3 < NCHUNK)
        def _prefetch1():
            _start_load(c + 3, 1)

    plsc.subcore_barrier()
    pltpu.sync_copy(shared.at[pl.ds(sid * N_STRIPE, N_STRIPE)],
                    out_hbm.at[cid].at[pl.ds(sid * N_STRIPE, N_STRIPE)])

    @pl.when(sid == NS - 1)
    def _out_tail():
        pltpu.sync_copy(shared.at[pl.ds(N_STRIPE * NS, N_TAIL)],
                        out_hbm.at[cid].at[pl.ds(N_STRIPE * NS, N_TAIL)])


_RB = 2000


def _mlp_body(x_ref, p_ref, wx_ref, we_ref, b_ref, o_ref):
    agg = p_ref[0] + p_ref[1]
    o_ref[...] = (
        jnp.dot(x_ref[...], wx_ref[...], preferred_element_type=jnp.float32)
        + jnp.dot(agg, we_ref[...], preferred_element_type=jnp.float32)
        + b_ref[...]
    )


def _mlp(x, parts, wx, we, b2):
    return pl.pallas_call(
        _mlp_body,
        grid=(N // _RB,),
        in_specs=[
            pl.BlockSpec((_RB, DF), lambda i: (i, 0)),
            pl.BlockSpec((NC, _RB, DE), lambda i: (0, i, 0)),
            pl.BlockSpec((DF, DF), lambda i: (0, 0)),
            pl.BlockSpec((DE, DF), lambda i: (0, 0)),
            pl.BlockSpec((1, DF), lambda i: (0, 0)),
        ],
        out_specs=pl.BlockSpec((_RB, DF), lambda i: (i, 0)),
        out_shape=jax.ShapeDtypeStruct((N, DF), jnp.float32),
    )(x, parts, wx, we, b2)


def kernel(x, edge_index, edge_attr, pos, W, b):
    zeros = jnp.zeros((N, DE), jnp.float32)
    parts = _sc_agg(edge_index, edge_attr.T, zeros)
    x_ = _mlp(x, parts, W[:DF], W[DF:], b.reshape(1, DF))
    return (x_, edge_attr, edge_index, pos)
